# baseline (speedup 1.0000x reference)
import jax
import jax.numpy as jnp
from jax.experimental import pallas as pl


def _linear_kernel(x_ref, wt_ref, b_ref, out_ref):
    xw = jnp.dot(
        x_ref[...].astype(jnp.bfloat16),
        wt_ref[...],
        preferred_element_type=jnp.float32,
    )
    out_ref[...] = (xw + b_ref[...]).astype(jnp.bfloat16)


def _agg_kernel(adj_ref, xw_ref, out_ref):
    out_ref[...] = jnp.dot(
        adj_ref[...].astype(jnp.bfloat16),
        xw_ref[...],
        preferred_element_type=jnp.float32,
    )


def kernel(x, adj, W, b):
    n, d_in = x.shape
    d_out = W.shape[0]
    wt_bf = W.T.astype(jnp.bfloat16)
    b2d = b.reshape(1, d_out)

    bm1 = 512
    xw = pl.pallas_call(
        _linear_kernel,
        grid=(pl.cdiv(n, bm1),),
        in_specs=[
            pl.BlockSpec((bm1, d_in), lambda i: (i, 0)),
            pl.BlockSpec((d_in, d_out), lambda i: (0, 0)),
            pl.BlockSpec((1, d_out), lambda i: (0, 0)),
        ],
        out_specs=pl.BlockSpec((bm1, d_out), lambda i: (i, 0)),
        out_shape=jax.ShapeDtypeStruct((n, d_out), jnp.bfloat16),
    )(x, wt_bf, b2d)

    bm2 = 256
    out = pl.pallas_call(
        _agg_kernel,
        grid=(pl.cdiv(n, bm2),),
        in_specs=[
            pl.BlockSpec((bm2, n), lambda i: (i, 0)),
            pl.BlockSpec((n, d_out), lambda i: (0, 0)),
        ],
        out_specs=pl.BlockSpec((bm2, d_out), lambda i: (i, 0)),
        out_shape=jax.ShapeDtypeStruct((n, d_out), jnp.float32),
    )(adj, xw)
    return out


# speedup vs baseline: 1.1045x; 1.1045x over previous
import jax
import jax.numpy as jnp
from jax.experimental import pallas as pl


def _linear_kernel(x_ref, wt_ref, b_ref, out_ref):
    xw = jnp.dot(
        x_ref[...].astype(jnp.bfloat16),
        wt_ref[...],
        preferred_element_type=jnp.float32,
    )
    out_ref[...] = (xw + b_ref[...]).astype(jnp.bfloat16)


def _agg_kernel(adj_ref, xw_ref, out_ref):
    out_ref[...] = jnp.dot(
        adj_ref[...].astype(jnp.bfloat16),
        xw_ref[...],
        preferred_element_type=jnp.float32,
    )


def kernel(x, adj, W, b):
    n, d_in = x.shape
    d_out = W.shape[0]
    wt_bf = W.T.astype(jnp.bfloat16)
    b2d = b.reshape(1, d_out)

    bm1 = 1024
    xw = pl.pallas_call(
        _linear_kernel,
        grid=(pl.cdiv(n, bm1),),
        in_specs=[
            pl.BlockSpec((bm1, d_in), lambda i: (i, 0)),
            pl.BlockSpec((d_in, d_out), lambda i: (0, 0)),
            pl.BlockSpec((1, d_out), lambda i: (0, 0)),
        ],
        out_specs=pl.BlockSpec((bm1, d_out), lambda i: (i, 0)),
        out_shape=jax.ShapeDtypeStruct((n, d_out), jnp.bfloat16),
    )(x, wt_bf, b2d)

    bm2 = 512
    out = pl.pallas_call(
        _agg_kernel,
        grid=(pl.cdiv(n, bm2),),
        in_specs=[
            pl.BlockSpec((bm2, n), lambda i: (i, 0)),
            pl.BlockSpec((n, d_out), lambda i: (0, 0)),
        ],
        out_specs=pl.BlockSpec((bm2, d_out), lambda i: (i, 0)),
        out_shape=jax.ShapeDtypeStruct((n, d_out), jnp.float32),
    )(adj, xw)
    return out


# speedup vs baseline: 1.2030x; 1.0892x over previous
import jax
import jax.numpy as jnp
from jax.experimental import pallas as pl
from jax.experimental.pallas import tpu as pltpu

_BM = 200


def _fused_kernel(x_ref, wt_ref, b_ref, adj_hbm, out_hbm, xw_ref):
    n_rows = x_ref.shape[0]
    chunk = 1250
    for c in range(n_rows // chunk):
        sl = pl.ds(c * chunk, chunk)
        xw = jnp.dot(
            x_ref[sl, :].astype(jnp.bfloat16),
            wt_ref[...],
            preferred_element_type=jnp.float32,
        )
        xw_ref[sl, :] = (xw + b_ref[...]).astype(jnp.bfloat16)

    n = adj_hbm.shape[0]
    d_out = out_hbm.shape[1]

    def _agg_body(adj_blk_ref, out_blk_ref):
        out_blk_ref[...] = jnp.dot(
            adj_blk_ref[...].astype(jnp.bfloat16),
            xw_ref[...],
            preferred_element_type=jnp.float32,
        )

    pipeline = pltpu.emit_pipeline(
        _agg_body,
        grid=(n // _BM,),
        in_specs=[
            pl.BlockSpec(
                (_BM, n),
                lambda i: (i, 0),
                pipeline_mode=pl.Buffered(buffer_count=3),
            )
        ],
        out_specs=[pl.BlockSpec((_BM, d_out), lambda i: (i, 0))],
    )
    pipeline(adj_hbm, out_hbm)


def kernel(x, adj, W, b):
    n, d_in = x.shape
    d_out = W.shape[0]
    wt_bf = W.T.astype(jnp.bfloat16)
    b2d = b.reshape(1, d_out)

    out = pl.pallas_call(
        _fused_kernel,
        in_specs=[
            pl.BlockSpec(memory_space=pltpu.VMEM),
            pl.BlockSpec(memory_space=pltpu.VMEM),
            pl.BlockSpec(memory_space=pltpu.VMEM),
            pl.BlockSpec(memory_space=pl.ANY),
        ],
        out_specs=pl.BlockSpec(memory_space=pl.ANY),
        out_shape=jax.ShapeDtypeStruct((n, d_out), jnp.float32),
        scratch_shapes=[pltpu.VMEM((n, d_out), jnp.bfloat16)],
    )(x, wt_bf, b2d, adj)
    return out


# speedup vs baseline: 1.2189x; 1.0132x over previous
import jax
import jax.numpy as jnp
from jax.experimental import pallas as pl
from jax.experimental.pallas import tpu as pltpu

_BM = 200


def _fused_kernel(wt_ref, b_ref, x_hbm, adj_hbm, out_hbm, xw_ref, xbuf_ref, xsem):
    n = adj_hbm.shape[0]
    d_out = out_hbm.shape[1]

    pltpu.make_async_copy(x_hbm, xbuf_ref, xsem).start()

    def _agg_body(adj_blk_ref, out_blk_ref):
        @pl.when(pl.program_id(0) == 0)
        def _stage1():
            pltpu.make_async_copy(x_hbm, xbuf_ref, xsem).wait()
            chunk = 1250
            for c in range(n // chunk):
                sl = pl.ds(c * chunk, chunk)
                xw = jnp.dot(
                    xbuf_ref[sl, :].astype(jnp.bfloat16),
                    wt_ref[...],
                    preferred_element_type=jnp.float32,
                )
                xw_ref[sl, :] = (xw + b_ref[...]).astype(jnp.bfloat16)

        out_blk_ref[...] = jnp.dot(
            adj_blk_ref[...].astype(jnp.bfloat16),
            xw_ref[...],
            preferred_element_type=jnp.float32,
        )

    pipeline = pltpu.emit_pipeline(
        _agg_body,
        grid=(n // _BM,),
        in_specs=[
            pl.BlockSpec(
                (_BM, n),
                lambda i: (i, 0),
                pipeline_mode=pl.Buffered(buffer_count=3),
            )
        ],
        out_specs=[pl.BlockSpec((_BM, d_out), lambda i: (i, 0))],
    )
    pipeline(adj_hbm, out_hbm)


def kernel(x, adj, W, b):
    n, d_in = x.shape
    d_out = W.shape[0]
    wt_bf = W.T.astype(jnp.bfloat16)
    b2d = b.reshape(1, d_out)

    out = pl.pallas_call(
        _fused_kernel,
        in_specs=[
            pl.BlockSpec(memory_space=pltpu.VMEM),
            pl.BlockSpec(memory_space=pltpu.VMEM),
            pl.BlockSpec(memory_space=pl.ANY),
            pl.BlockSpec(memory_space=pl.ANY),
        ],
        out_specs=pl.BlockSpec(memory_space=pl.ANY),
        out_shape=jax.ShapeDtypeStruct((n, d_out), jnp.float32),
        scratch_shapes=[
            pltpu.VMEM((n, d_out), jnp.bfloat16),
            pltpu.VMEM((n, d_in), jnp.float32),
            pltpu.SemaphoreType.DMA,
        ],
    )(wt_bf, b2d, x, adj)
    return out
